# baseline (speedup 1.0000x reference)
import jax
import jax.numpy as jnp
from jax import lax
from jax.experimental import pallas as pl
from jax.experimental.pallas import tpu as pltpu
from jax.experimental.pallas import tpu_sc as plsc

_FEAT = 128
_BATCH = 16384
_LAMDA = 1.0
_SCALE = 1.0
_NC = 2
_NS = 16
_NW = _NC * _NS
_RPW = _BATCH // _NW
_CHUNK = 64
_NCHUNK = _RPW // _CHUNK
_NBUF = 3
_LANES = 16
_JG = _FEAT // _LANES


def _sc_body(x_hbm, y_hbm, table_hbm, out_hbm,
             idx_v, feat_v, rows_v, acc_v,
             sem_f0, sem_f1, sem_g0, sem_g1, sem_g2, sem_g3,
             sem_g4, sem_g5, sem_g6, sem_g7):
  cid = lax.axis_index("c")
  sid = lax.axis_index("s")
  wid = cid * _NS + sid
  sem_g = (sem_g0, sem_g1, sem_g2, sem_g3, sem_g4, sem_g5, sem_g6, sem_g7)
  sem_f = (sem_f0, sem_f1)

  def start_feat(c):
    return pltpu.async_copy(x_hbm.at[wid, c], feat_v.at[c % 2], sem_f[c % 2])

  pf = start_feat(0)

  pltpu.sync_copy(y_hbm.at[wid, 0], idx_v.at[0])
  pg = [pltpu.async_copy(table_hbm.at[idx_v.at[0]], rows_v.at[0], sem_g[0])]
  pltpu.sync_copy(y_hbm.at[wid, pl.ds(1, _NCHUNK - 1)],
                  idx_v.at[pl.ds(1, _NCHUNK - 1)])

  pg += [pltpu.async_copy(table_hbm.at[idx_v.at[c]], rows_v.at[c], sem_g[c])
         for c in range(1, _NCHUNK)]

  accs = tuple(jnp.zeros((_LANES,), jnp.float32) for _ in range(_JG))
  for c in range(_NCHUNK):
    pf.wait()
    if c + 1 < _NCHUNK:
      pf = start_feat(c + 1)
    pg[c].wait()

    @plsc.parallel_loop(0, _CHUNK, carry=accs, unroll=16)
    def _row(r, a):
      out = []
      for j in range(_JG):
        d = (feat_v[c % 2, r, pl.ds(j * _LANES, _LANES)]
             - rows_v[c, r, pl.ds(j * _LANES, _LANES)])
        out.append(a[j] + d * d)
      return tuple(out)

    accs = _row

  total = accs[0]
  for j in range(1, _JG):
    total = total + accs[j]

  s = total[0]
  for i in range(1, _LANES):
    s = s + total[i]
  acc_v[...] = jnp.full((_LANES,), s, jnp.float32)
  pltpu.sync_copy(acc_v, out_hbm.at[wid])


def kernel(output_features, y_truth, feature_centers):
  x = output_features.reshape(_NW, _NCHUNK, _CHUNK, _FEAT)
  y = y_truth.astype(jnp.int32).reshape(_NW, _NCHUNK, _CHUNK)

  mesh = plsc.VectorSubcoreMesh(core_axis_name="c", subcore_axis_name="s")
  out = pl.kernel(
      _sc_body,
      out_type=jax.ShapeDtypeStruct((_NW, _LANES), jnp.float32),
      mesh=mesh,
      scratch_types=[
          pltpu.VMEM((_NCHUNK, _CHUNK), jnp.int32),
          pltpu.VMEM((2, _CHUNK, _FEAT), jnp.float32),
          pltpu.VMEM((_NCHUNK, _CHUNK, _FEAT), jnp.float32),
          pltpu.VMEM((_LANES,), jnp.float32),
          pltpu.SemaphoreType.DMA,
          pltpu.SemaphoreType.DMA,
          pltpu.SemaphoreType.DMA,
          pltpu.SemaphoreType.DMA,
          pltpu.SemaphoreType.DMA,
          pltpu.SemaphoreType.DMA,
          pltpu.SemaphoreType.DMA,
          pltpu.SemaphoreType.DMA,
          pltpu.SemaphoreType.DMA,
          pltpu.SemaphoreType.DMA,
      ],
  )(x, y, feature_centers)

  factor = _LAMDA * 0.5 * _SCALE / _BATCH
  return jnp.sum(out[:, 0]) * jnp.float32(factor)


# speedup vs baseline: 1.0712x; 1.0712x over previous
import jax
import jax.numpy as jnp
from jax import lax
from jax.experimental import pallas as pl
from jax.experimental.pallas import tpu as pltpu
from jax.experimental.pallas import tpu_sc as plsc

_FEAT = 128
_BATCH = 16384
_LAMDA = 1.0
_SCALE = 1.0
_NC = 2
_NS = 16
_NW = _NC * _NS
_RPW = _BATCH // _NW
_CHUNK = 128
_NCHUNK = _RPW // _CHUNK
_LANES = 16
_JG = _FEAT // _LANES


def _sc_body(x_hbm, y_hbm, table_hbm, out_hbm,
             idx_v, feat_v, rows_v, acc_v,
             sem_f0, sem_f1, sem_g0, sem_g1, sem_g2, sem_g3):
  cid = lax.axis_index("c")
  sid = lax.axis_index("s")
  wid = cid * _NS + sid
  sem_g = (sem_g0, sem_g1, sem_g2, sem_g3)
  sem_f = (sem_f0, sem_f1)

  def start_feat(c):
    return pltpu.async_copy(x_hbm.at[wid, c], feat_v.at[c % 2], sem_f[c % 2])

  pf = start_feat(0)

  pltpu.sync_copy(y_hbm.at[wid, 0], idx_v.at[0])
  pg = [pltpu.async_copy(table_hbm.at[idx_v.at[0]], rows_v.at[0], sem_g[0])]
  pltpu.sync_copy(y_hbm.at[wid, pl.ds(1, _NCHUNK - 1)],
                  idx_v.at[pl.ds(1, _NCHUNK - 1)])

  pg += [pltpu.async_copy(table_hbm.at[idx_v.at[c]], rows_v.at[c], sem_g[c])
         for c in range(1, _NCHUNK)]

  accs = tuple(jnp.zeros((_LANES,), jnp.float32) for _ in range(_JG))
  for c in range(_NCHUNK):
    pf.wait()
    if c + 1 < _NCHUNK:
      pf = start_feat(c + 1)
    pg[c].wait()

    @plsc.parallel_loop(0, _CHUNK, carry=accs, unroll=16)
    def _row(r, a):
      out = []
      for j in range(_JG):
        d = (feat_v[c % 2, r, pl.ds(j * _LANES, _LANES)]
             - rows_v[c, r, pl.ds(j * _LANES, _LANES)])
        out.append(a[j] + d * d)
      return tuple(out)

    accs = _row

  total = accs[0]
  for j in range(1, _JG):
    total = total + accs[j]

  s = total[0]
  for i in range(1, _LANES):
    s = s + total[i]
  acc_v[...] = jnp.full((_LANES,), s, jnp.float32)
  pltpu.sync_copy(acc_v, out_hbm.at[wid])


def kernel(output_features, y_truth, feature_centers):
  x = output_features.reshape(_NW, _NCHUNK, _CHUNK, _FEAT)
  y = y_truth.astype(jnp.int32).reshape(_NW, _NCHUNK, _CHUNK)

  mesh = plsc.VectorSubcoreMesh(core_axis_name="c", subcore_axis_name="s")
  out = pl.kernel(
      _sc_body,
      out_type=jax.ShapeDtypeStruct((_NW, _LANES), jnp.float32),
      mesh=mesh,
      scratch_types=[
          pltpu.VMEM((_NCHUNK, _CHUNK), jnp.int32),
          pltpu.VMEM((2, _CHUNK, _FEAT), jnp.float32),
          pltpu.VMEM((_NCHUNK, _CHUNK, _FEAT), jnp.float32),
          pltpu.VMEM((_LANES,), jnp.float32),
          pltpu.SemaphoreType.DMA,
          pltpu.SemaphoreType.DMA,
          pltpu.SemaphoreType.DMA,
          pltpu.SemaphoreType.DMA,
          pltpu.SemaphoreType.DMA,
          pltpu.SemaphoreType.DMA,
      ],
  )(x, y, feature_centers)

  factor = _LAMDA * 0.5 * _SCALE / _BATCH
  return jnp.sum(out[:, 0]) * jnp.float32(factor)


# speedup vs baseline: 1.0728x; 1.0016x over previous
import jax
import jax.numpy as jnp
from jax import lax
from jax.experimental import pallas as pl
from jax.experimental.pallas import tpu as pltpu
from jax.experimental.pallas import tpu_sc as plsc

_FEAT = 128
_BATCH = 16384
_LAMDA = 1.0
_SCALE = 1.0
_NC = 2
_NS = 16
_NW = _NC * _NS
_RPW = _BATCH // _NW
_CHUNK = 128
_NCHUNK = _RPW // _CHUNK
_LANES = 16
_JG = _FEAT // _LANES


def _sc_body(x_hbm, y_hbm, table_hbm, out_hbm,
             idx_v, feat_v, rows_v, acc_v,
             sem_f0, sem_f1, sem_g0, sem_g1, sem_g2, sem_g3):
  cid = lax.axis_index("c")
  sid = lax.axis_index("s")
  wid = cid * _NS + sid
  sem_g = (sem_g0, sem_g1, sem_g2, sem_g3)
  sem_f = (sem_f0, sem_f1)

  def start_feat(c):
    return pltpu.async_copy(x_hbm.at[wid, c], feat_v.at[c % 2], sem_f[c % 2])

  pf = start_feat(0)

  pltpu.sync_copy(y_hbm.at[wid, 0], idx_v.at[0])
  pg = [pltpu.async_copy(table_hbm.at[idx_v.at[0]], rows_v.at[0], sem_g[0])]
  pltpu.sync_copy(y_hbm.at[wid, pl.ds(1, _NCHUNK - 1)],
                  idx_v.at[pl.ds(1, _NCHUNK - 1)])

  pg += [pltpu.async_copy(table_hbm.at[idx_v.at[c]], rows_v.at[c], sem_g[c])
         for c in range(1, _NCHUNK)]

  accs = tuple(jnp.zeros((_LANES,), jnp.float32) for _ in range(_JG))
  for c in range(_NCHUNK):
    pf.wait()
    if c + 1 < _NCHUNK:
      pf = start_feat(c + 1)
    pg[c].wait()

    @plsc.parallel_loop(0, _CHUNK, carry=accs, unroll=32)
    def _row(r, a):
      out = []
      for j in range(_JG):
        d = (feat_v[c % 2, r, pl.ds(j * _LANES, _LANES)]
             - rows_v[c, r, pl.ds(j * _LANES, _LANES)])
        out.append(a[j] + d * d)
      return tuple(out)

    accs = _row

  total = accs[0]
  for j in range(1, _JG):
    total = total + accs[j]

  s = total[0]
  for i in range(1, _LANES):
    s = s + total[i]
  acc_v[...] = jnp.full((_LANES,), s, jnp.float32)
  pltpu.sync_copy(acc_v, out_hbm.at[wid])


def kernel(output_features, y_truth, feature_centers):
  x = output_features.reshape(_NW, _NCHUNK, _CHUNK, _FEAT)
  y = y_truth.astype(jnp.int32).reshape(_NW, _NCHUNK, _CHUNK)

  mesh = plsc.VectorSubcoreMesh(core_axis_name="c", subcore_axis_name="s")
  out = pl.kernel(
      _sc_body,
      out_type=jax.ShapeDtypeStruct((_NW, _LANES), jnp.float32),
      mesh=mesh,
      scratch_types=[
          pltpu.VMEM((_NCHUNK, _CHUNK), jnp.int32),
          pltpu.VMEM((2, _CHUNK, _FEAT), jnp.float32),
          pltpu.VMEM((_NCHUNK, _CHUNK, _FEAT), jnp.float32),
          pltpu.VMEM((_LANES,), jnp.float32),
          pltpu.SemaphoreType.DMA,
          pltpu.SemaphoreType.DMA,
          pltpu.SemaphoreType.DMA,
          pltpu.SemaphoreType.DMA,
          pltpu.SemaphoreType.DMA,
          pltpu.SemaphoreType.DMA,
      ],
  )(x, y, feature_centers)

  factor = _LAMDA * 0.5 * _SCALE / _BATCH
  return jnp.sum(out[:, 0]) * jnp.float32(factor)


# speedup vs baseline: 1.0780x; 1.0049x over previous
import jax
import jax.numpy as jnp
from jax import lax
from jax.experimental import pallas as pl
from jax.experimental.pallas import tpu as pltpu
from jax.experimental.pallas import tpu_sc as plsc

_FEAT = 128
_BATCH = 16384
_LAMDA = 1.0
_SCALE = 1.0
_NC = 2
_NS = 16
_NW = _NC * _NS
_RPW = _BATCH // _NW
_CHUNK = 128
_NCHUNK = _RPW // _CHUNK
_LANES = 16
_JG = _FEAT // _LANES


def _sc_body(x_hbm, y_hbm, table_hbm, out_hbm,
             idx_v, feat_v, rows_v, acc_v,
             sem_f0, sem_f1, sem_g0, sem_g1, sem_g2, sem_g3):
  cid = lax.axis_index("c")
  sid = lax.axis_index("s")
  wid = cid * _NS + sid
  sem_g = (sem_g0, sem_g1, sem_g2, sem_g3)
  sem_f = (sem_f0, sem_f1)

  def start_feat(c):
    return pltpu.async_copy(x_hbm.at[wid, c], feat_v.at[c % 2], sem_f[c % 2])

  pf = start_feat(0)

  pltpu.sync_copy(y_hbm.at[wid, 0], idx_v.at[0])
  pg = [pltpu.async_copy(table_hbm.at[idx_v.at[0]], rows_v.at[0], sem_g[0])]
  pltpu.sync_copy(y_hbm.at[wid, pl.ds(1, _NCHUNK - 1)],
                  idx_v.at[pl.ds(1, _NCHUNK - 1)])

  pg += [pltpu.async_copy(table_hbm.at[idx_v.at[c]], rows_v.at[c], sem_g[c])
         for c in range(1, _NCHUNK)]

  accs = tuple(jnp.zeros((_LANES,), jnp.float32) for _ in range(_JG))
  for c in range(_NCHUNK):
    pf.wait()
    if c + 1 < _NCHUNK:
      pf = start_feat(c + 1)
    pg[c].wait()

    @plsc.parallel_loop(0, _CHUNK, carry=accs, unroll=16)
    def _row(r, a):
      out = []
      for j in range(_JG):
        d = (feat_v[c % 2, r, pl.ds(j * _LANES, _LANES)]
             - rows_v[c, r, pl.ds(j * _LANES, _LANES)])
        out.append(a[j] + d * d)
      return tuple(out)

    accs = _row

  total = accs[0]
  for j in range(1, _JG):
    total = total + accs[j]

  s = total[0]
  for i in range(1, _LANES):
    s = s + total[i]
  acc_v[...] = jnp.full((_LANES,), s, jnp.float32)
  pltpu.sync_copy(acc_v, out_hbm.at[wid])


def kernel(output_features, y_truth, feature_centers):
  x = output_features.reshape(_NW, _NCHUNK, _CHUNK, _FEAT)
  y = y_truth.astype(jnp.int32).reshape(_NW, _NCHUNK, _CHUNK)

  mesh = plsc.VectorSubcoreMesh(core_axis_name="c", subcore_axis_name="s")
  out = pl.kernel(
      _sc_body,
      out_type=jax.ShapeDtypeStruct((_NW, _LANES), jnp.float32),
      mesh=mesh,
      scratch_types=[
          pltpu.VMEM((_NCHUNK, _CHUNK), jnp.int32),
          pltpu.VMEM((2, _CHUNK, _FEAT), jnp.float32),
          pltpu.VMEM((_NCHUNK, _CHUNK, _FEAT), jnp.float32),
          pltpu.VMEM((_LANES,), jnp.float32),
          pltpu.SemaphoreType.DMA,
          pltpu.SemaphoreType.DMA,
          pltpu.SemaphoreType.DMA,
          pltpu.SemaphoreType.DMA,
          pltpu.SemaphoreType.DMA,
          pltpu.SemaphoreType.DMA,
      ],
  )(x, y, feature_centers)

  factor = _LAMDA * 0.5 * _SCALE / _BATCH
  return jnp.sum(out[:, 0]) * jnp.float32(factor)
